# baseline (speedup 1.0000x reference)
import functools

import jax
import jax.numpy as jnp
from jax import lax
from jax.experimental import pallas as pl
from jax.experimental.pallas import tpu as pltpu
from jax.experimental.pallas import tpu_sc as plsc

N = 10000
E = 320000
D = 128
DH = 64
H = 128
C = 41
CP = 48

NC = 2
NS = 16
NW = NC * NS
K = 128
EP = 327680
CH1 = EP // (NS * K)
CH2 = EP // (NW * K)
NPAD = 10112
RPT = NPAD // NS


def _pass1_body(src_hbm, dst_hbm, xs_hbm, zrows_hbm, zvec_hbm, ones_hbm,
                s_out, deg_out,
                src_v, dsta_v, dstb_v, rows0_v, rows1_v, ones_v, degb_v,
                x_sh, acc_sh, deg_sh, gsem0, gsem1, isema, isemb, ssem0):
  c = lax.axis_index("c")
  s = lax.axis_index("s")
  sl = pl.ds(s * RPT, RPT)

  pltpu.sync_copy(zrows_hbm, acc_sh.at[sl])
  pltpu.sync_copy(xs_hbm.at[sl, pl.ds(c * DH, DH)], x_sh.at[sl])
  pltpu.sync_copy(src_hbm.at[s], src_v)
  pltpu.sync_copy(ones_hbm, ones_v)
  pltpu.sync_copy(zvec_hbm, degb_v)
  pltpu.sync_copy(degb_v, deg_sh.at[sl])
  plsc.subcore_barrier()

  pltpu.async_copy(dst_hbm.at[s, 0], dsta_v, isema)
  pltpu.async_copy(dst_hbm.at[s, 1], dstb_v, isemb)
  pltpu.async_copy(x_sh.at[src_v.at[0]], rows0_v, gsem0)

  def body(i, _):
    j0 = 2 * i
    j1 = j0 + 1
    j2 = jnp.minimum(j0 + 2, CH1 - 1)
    j3 = jnp.minimum(j0 + 3, CH1 - 1)
    pltpu.async_copy(x_sh.at[src_v.at[j1]], rows1_v, gsem1)
    pltpu.make_async_copy(x_sh.at[src_v.at[j0]], rows0_v, gsem0).wait()
    pltpu.make_async_copy(dst_hbm.at[s, j0], dsta_v, isema).wait()
    scat0 = pltpu.async_copy(rows0_v, acc_sh.at[dsta_v], ssem0, add=True)

    @pl.when(c == 0)
    def _():
      pltpu.sync_copy(ones_v, deg_sh.at[dsta_v], add=True)

    pltpu.make_async_copy(x_sh.at[src_v.at[j1]], rows1_v, gsem1).wait()
    pltpu.make_async_copy(dst_hbm.at[s, j1], dstb_v, isemb).wait()
    pltpu.sync_copy(rows1_v, acc_sh.at[dstb_v], add=True)

    @pl.when(c == 1)
    def _():
      pltpu.sync_copy(ones_v, deg_sh.at[dstb_v], add=True)

    scat0.wait()
    pltpu.async_copy(x_sh.at[src_v.at[j2]], rows0_v, gsem0)
    pltpu.async_copy(dst_hbm.at[s, j2], dsta_v, isema)
    pltpu.async_copy(dst_hbm.at[s, j3], dstb_v, isemb)
    return 0

  lax.fori_loop(0, CH1 // 2, body, 0)
  pltpu.make_async_copy(x_sh.at[src_v.at[CH1 - 1]], rows0_v, gsem0).wait()
  pltpu.make_async_copy(dst_hbm.at[s, CH1 - 1], dsta_v, isema).wait()
  pltpu.make_async_copy(dst_hbm.at[s, CH1 - 1], dstb_v, isemb).wait()
  plsc.subcore_barrier()

  pltpu.sync_copy(acc_sh.at[sl], s_out.at[c, sl])
  pltpu.sync_copy(deg_sh.at[sl], degb_v)
  pltpu.sync_copy(degb_v, deg_out.at[pl.ds(c * NPAD + s * RPT, RPT)])


def _make_pass1():
  mesh = plsc.VectorSubcoreMesh(core_axis_name="c", subcore_axis_name="s")
  out_type = [jax.ShapeDtypeStruct((NC, NPAD, DH), jnp.float32),
              jax.ShapeDtypeStruct((NC * NPAD,), jnp.float32)]
  scratch = [
      pltpu.VMEM((CH1, K), jnp.int32),
      pltpu.VMEM((K,), jnp.int32),
      pltpu.VMEM((K,), jnp.int32),
      pltpu.VMEM((K, DH), jnp.float32),
      pltpu.VMEM((K, DH), jnp.float32),
      pltpu.VMEM((K,), jnp.float32),
      pltpu.VMEM((RPT,), jnp.float32),
      pltpu.VMEM_SHARED((NPAD, DH), jnp.float32),
      pltpu.VMEM_SHARED((NPAD, DH), jnp.float32),
      pltpu.VMEM_SHARED((NPAD,), jnp.float32),
      pltpu.SemaphoreType.DMA,
      pltpu.SemaphoreType.DMA,
      pltpu.SemaphoreType.DMA,
      pltpu.SemaphoreType.DMA,
      pltpu.SemaphoreType.DMA,
  ]
  return pl.kernel(
      _pass1_body, out_type=out_type, mesh=mesh, scratch_types=scratch,
      compiler_params=pltpu.CompilerParams(use_tc_tiling_on_sc=False))


def _pass2_body(src_hbm, dst_hbm, hp_hbm, zrows_hbm,
                s_out,
                src_v, dsta_v, dstb_v, rows0_v, rows1_v, hp_sh, acc_sh,
                gsem0, gsem1, isema, isemb, ssem0):
  c = lax.axis_index("c")
  s = lax.axis_index("s")
  wid = s * NC + c
  sl = pl.ds(s * RPT, RPT)

  pltpu.sync_copy(zrows_hbm, acc_sh.at[sl])
  pltpu.sync_copy(hp_hbm.at[sl], hp_sh.at[sl])
  pltpu.sync_copy(src_hbm.at[wid], src_v)
  plsc.subcore_barrier()

  pltpu.async_copy(dst_hbm.at[wid, 0], dsta_v, isema)
  pltpu.async_copy(dst_hbm.at[wid, 1], dstb_v, isemb)
  pltpu.async_copy(hp_sh.at[src_v.at[0]], rows0_v, gsem0)

  def body(i, _):
    j0 = 2 * i
    j1 = j0 + 1
    j2 = jnp.minimum(j0 + 2, CH2 - 1)
    j3 = jnp.minimum(j0 + 3, CH2 - 1)
    pltpu.async_copy(hp_sh.at[src_v.at[j1]], rows1_v, gsem1)
    pltpu.make_async_copy(hp_sh.at[src_v.at[j0]], rows0_v, gsem0).wait()
    pltpu.make_async_copy(dst_hbm.at[wid, j0], dsta_v, isema).wait()
    scat0 = pltpu.async_copy(rows0_v, acc_sh.at[dsta_v], ssem0, add=True)
    pltpu.make_async_copy(hp_sh.at[src_v.at[j1]], rows1_v, gsem1).wait()
    pltpu.make_async_copy(dst_hbm.at[wid, j1], dstb_v, isemb).wait()
    pltpu.sync_copy(rows1_v, acc_sh.at[dstb_v], add=True)
    scat0.wait()
    pltpu.async_copy(hp_sh.at[src_v.at[j2]], rows0_v, gsem0)
    pltpu.async_copy(dst_hbm.at[wid, j2], dsta_v, isema)
    pltpu.async_copy(dst_hbm.at[wid, j3], dstb_v, isemb)
    return 0

  lax.fori_loop(0, CH2 // 2, body, 0)
  pltpu.make_async_copy(hp_sh.at[src_v.at[CH2 - 1]], rows0_v, gsem0).wait()
  pltpu.make_async_copy(dst_hbm.at[wid, CH2 - 1], dsta_v, isema).wait()
  pltpu.make_async_copy(dst_hbm.at[wid, CH2 - 1], dstb_v, isemb).wait()
  plsc.subcore_barrier()

  pltpu.sync_copy(acc_sh.at[sl], s_out.at[c, sl])


def _make_pass2():
  mesh = plsc.VectorSubcoreMesh(core_axis_name="c", subcore_axis_name="s")
  out_type = [jax.ShapeDtypeStruct((NC, NPAD, CP), jnp.float32)]
  scratch = [
      pltpu.VMEM((CH2, K), jnp.int32),
      pltpu.VMEM((K,), jnp.int32),
      pltpu.VMEM((K,), jnp.int32),
      pltpu.VMEM((K, CP), jnp.float32),
      pltpu.VMEM((K, CP), jnp.float32),
      pltpu.VMEM_SHARED((NPAD, CP), jnp.float32),
      pltpu.VMEM_SHARED((NPAD, CP), jnp.float32),
      pltpu.SemaphoreType.DMA,
      pltpu.SemaphoreType.DMA,
      pltpu.SemaphoreType.DMA,
      pltpu.SemaphoreType.DMA,
      pltpu.SemaphoreType.DMA,
  ]
  return pl.kernel(
      _pass2_body, out_type=out_type, mesh=mesh, scratch_types=scratch,
      compiler_params=pltpu.CompilerParams(use_tc_tiling_on_sc=False))


def _tc1_body(s1a, s1b, dega, degb, x, w1l, w1r, b1, w2lp, h_out, hp_out):
  d = jnp.maximum(dega[0] + degb[0], 1.0)
  agg = jnp.concatenate([s1a[0], s1b[0]], axis=1) / d
  h = agg @ w1l[...] + x[...] @ w1r[...] + b1[...]
  h = jnp.maximum(h, 0.0)
  h_out[...] = h
  hp_out[...] = h @ w2lp[...]


def _tc2_body(s2a, s2b, dega, degb, h, w2rp, b2p, out):
  d = jnp.maximum(dega[0] + degb[0], 1.0)
  logits = (s2a[0] + s2b[0]) / d + h[...] @ w2rp[...] + b2p[...]
  col = lax.broadcasted_iota(jnp.int32, logits.shape, 1)
  ml = jnp.where(col < C, logits, -1e30)
  m = jnp.max(ml, axis=-1, keepdims=True)
  lse = jnp.log(jnp.sum(jnp.exp(ml - m), axis=-1, keepdims=True)) + m
  out[...] = ml - lse


_BR = 1264


def _tc1(s1, deg, xpad, w1l, w1r, b1, w2lp):
  grid = (NPAD // _BR,)
  row = lambda i: (i, 0)
  rowa = lambda i: (0, i, 0)
  rowb = lambda i: (1, i, 0)
  full = lambda i: (0, 0)
  return pl.pallas_call(
      _tc1_body,
      grid=grid,
      in_specs=[
          pl.BlockSpec((1, _BR, DH), rowa), pl.BlockSpec((1, _BR, DH), rowb),
          pl.BlockSpec((1, _BR, 1), rowa), pl.BlockSpec((1, _BR, 1), rowb),
          pl.BlockSpec((_BR, D), row),
          pl.BlockSpec((D, H), full), pl.BlockSpec((D, H), full),
          pl.BlockSpec((1, H), full), pl.BlockSpec((H, CP), full),
      ],
      out_specs=[pl.BlockSpec((_BR, H), row), pl.BlockSpec((_BR, CP), row)],
      out_shape=[jax.ShapeDtypeStruct((NPAD, H), jnp.float32),
                 jax.ShapeDtypeStruct((NPAD, CP), jnp.float32)],
  )(s1, s1, deg, deg, xpad, w1l, w1r, b1, w2lp)


def _tc2(s2, deg, h, w2rp, b2p):
  grid = (NPAD // _BR,)
  row = lambda i: (i, 0)
  rowa = lambda i: (0, i, 0)
  rowb = lambda i: (1, i, 0)
  full = lambda i: (0, 0)
  return pl.pallas_call(
      _tc2_body,
      grid=grid,
      in_specs=[
          pl.BlockSpec((1, _BR, CP), rowa), pl.BlockSpec((1, _BR, CP), rowb),
          pl.BlockSpec((1, _BR, 1), rowa), pl.BlockSpec((1, _BR, 1), rowb),
          pl.BlockSpec((_BR, H), row),
          pl.BlockSpec((H, CP), full), pl.BlockSpec((1, CP), full),
      ],
      out_specs=pl.BlockSpec((_BR, CP), row),
      out_shape=jax.ShapeDtypeStruct((NPAD, CP), jnp.float32),
  )(s2, s2, deg, deg, h, w2rp, b2p)


def kernel(x, edge_index, W1l, W1r, b1, W2l, W2r, b2):
  src = edge_index[0].astype(jnp.int32)
  dst = edge_index[1].astype(jnp.int32)
  pad = EP - E
  srcp = jnp.concatenate([src, jnp.zeros((pad,), jnp.int32)])
  dstp = jnp.concatenate([dst, jnp.full((pad,), N, jnp.int32)])
  src1 = srcp.reshape(NS, CH1, K)
  dst1 = dstp.reshape(NS, CH1, K)
  src2 = srcp.reshape(NW, CH2, K)
  dst2 = dstp.reshape(NW, CH2, K)

  xpad = jnp.pad(x, ((0, NPAD - N), (0, 0)))

  zrows = jnp.zeros((RPT, DH), jnp.float32)
  zrows_c = jnp.zeros((RPT, CP), jnp.float32)
  zvec = jnp.zeros((RPT,), jnp.float32)
  ones = jnp.ones((K,), jnp.float32)

  s1, deg = _make_pass1()(src1, dst1, xpad, zrows, zvec, ones)
  deg = deg.reshape(NC, NPAD, 1)

  w2lp = jnp.pad(W2l, ((0, 0), (0, CP - C)))
  h, hp = _tc1(s1, deg, xpad, W1l, W1r, b1[None, :], w2lp)

  s2 = _make_pass2()(src2, dst2, hp, zrows_c)
  if isinstance(s2, (tuple, list)):
    s2 = s2[0]

  w2rp = jnp.pad(W2r, ((0, 0), (0, CP - C)))
  b2p = jnp.pad(b2, (0, CP - C))[None, :]
  out = _tc2(s2, deg, h, w2rp, b2p)
  return out[:N, :C]


# speedup vs baseline: 1.2210x; 1.2210x over previous
import functools

import jax
import jax.numpy as jnp
from jax import lax
from jax.experimental import pallas as pl
from jax.experimental.pallas import tpu as pltpu
from jax.experimental.pallas import tpu_sc as plsc

N = 10000
E = 320000
D = 128
DH = 64
H = 128
C = 41
CP = 48

NC = 2
NS = 16
NW = NC * NS
K = 128
EP = 327680
CH1 = EP // (NS * K)
CH2 = EP // (NW * K)
NPAD = 10112
RPT = NPAD // NS


def _pass1_body(src_hbm, dst_hbm, xs_hbm, zrows_hbm, zvec_hbm, ones_hbm,
                s_out, deg_out,
                src_v, dsta_v, dstb_v, rows0_v, rows1_v, ones_v, degb_v,
                x_sh, acc_sh, deg_sh, gsem0, gsem1, isema, isemb, ssem0):
  c = lax.axis_index("c")
  s = lax.axis_index("s")
  sl = pl.ds(s * RPT, RPT)

  pltpu.sync_copy(zrows_hbm, acc_sh.at[sl])
  pltpu.sync_copy(xs_hbm.at[sl, pl.ds(c * DH, DH)], x_sh.at[sl])
  pltpu.sync_copy(src_hbm.at[s], src_v)
  pltpu.sync_copy(ones_hbm, ones_v)
  pltpu.sync_copy(zvec_hbm, degb_v)
  pltpu.sync_copy(degb_v, deg_sh.at[sl])
  plsc.subcore_barrier()

  pltpu.async_copy(dst_hbm.at[s, 0], dsta_v, isema)
  pltpu.async_copy(dst_hbm.at[s, 1], dstb_v, isemb)
  pltpu.async_copy(x_sh.at[src_v.at[0]], rows0_v, gsem0)

  def body(i, _):
    j0 = 2 * i
    j1 = j0 + 1
    j2 = jnp.minimum(j0 + 2, CH1 - 1)
    j3 = jnp.minimum(j0 + 3, CH1 - 1)
    pltpu.async_copy(x_sh.at[src_v.at[j1]], rows1_v, gsem1)
    pltpu.make_async_copy(x_sh.at[src_v.at[j0]], rows0_v, gsem0).wait()
    pltpu.make_async_copy(dst_hbm.at[s, j0], dsta_v, isema).wait()
    pltpu.sync_copy(rows0_v, acc_sh.at[dsta_v], add=True)

    @pl.when(c == 0)
    def _():
      pltpu.sync_copy(ones_v, deg_sh.at[dsta_v], add=True)

    pltpu.async_copy(x_sh.at[src_v.at[j2]], rows0_v, gsem0)
    pltpu.async_copy(dst_hbm.at[s, j2], dsta_v, isema)
    pltpu.make_async_copy(x_sh.at[src_v.at[j1]], rows1_v, gsem1).wait()
    pltpu.make_async_copy(dst_hbm.at[s, j1], dstb_v, isemb).wait()
    pltpu.sync_copy(rows1_v, acc_sh.at[dstb_v], add=True)

    @pl.when(c == 1)
    def _():
      pltpu.sync_copy(ones_v, deg_sh.at[dstb_v], add=True)

    pltpu.async_copy(dst_hbm.at[s, j3], dstb_v, isemb)
    return 0

  lax.fori_loop(0, CH1 // 2, body, 0)
  pltpu.make_async_copy(x_sh.at[src_v.at[CH1 - 1]], rows0_v, gsem0).wait()
  pltpu.make_async_copy(dst_hbm.at[s, CH1 - 1], dsta_v, isema).wait()
  pltpu.make_async_copy(dst_hbm.at[s, CH1 - 1], dstb_v, isemb).wait()
  plsc.subcore_barrier()

  pltpu.sync_copy(acc_sh.at[sl], s_out.at[c, sl])
  pltpu.sync_copy(deg_sh.at[sl], degb_v)
  pltpu.sync_copy(degb_v, deg_out.at[pl.ds(c * NPAD + s * RPT, RPT)])


def _make_pass1():
  mesh = plsc.VectorSubcoreMesh(core_axis_name="c", subcore_axis_name="s")
  out_type = [jax.ShapeDtypeStruct((NC, NPAD, DH), jnp.float32),
              jax.ShapeDtypeStruct((NC * NPAD,), jnp.float32)]
  scratch = [
      pltpu.VMEM((CH1, K), jnp.int32),
      pltpu.VMEM((K,), jnp.int32),
      pltpu.VMEM((K,), jnp.int32),
      pltpu.VMEM((K, DH), jnp.float32),
      pltpu.VMEM((K, DH), jnp.float32),
      pltpu.VMEM((K,), jnp.float32),
      pltpu.VMEM((RPT,), jnp.float32),
      pltpu.VMEM_SHARED((NPAD, DH), jnp.float32),
      pltpu.VMEM_SHARED((NPAD, DH), jnp.float32),
      pltpu.VMEM_SHARED((NPAD,), jnp.float32),
      pltpu.SemaphoreType.DMA,
      pltpu.SemaphoreType.DMA,
      pltpu.SemaphoreType.DMA,
      pltpu.SemaphoreType.DMA,
      pltpu.SemaphoreType.DMA,
  ]
  return pl.kernel(
      _pass1_body, out_type=out_type, mesh=mesh, scratch_types=scratch,
      compiler_params=pltpu.CompilerParams(use_tc_tiling_on_sc=False))


def _pass2_body(src_hbm, dst_hbm, hp_hbm, zrows_hbm,
                s_out,
                src_v, dsta_v, dstb_v, rows0_v, rows1_v, hp_sh, acc_sh,
                gsem0, gsem1, isema, isemb, ssem0):
  c = lax.axis_index("c")
  s = lax.axis_index("s")
  wid = s * NC + c
  sl = pl.ds(s * RPT, RPT)

  pltpu.sync_copy(zrows_hbm, acc_sh.at[sl])
  pltpu.sync_copy(hp_hbm.at[sl], hp_sh.at[sl])
  pltpu.sync_copy(src_hbm.at[wid], src_v)
  plsc.subcore_barrier()

  pltpu.async_copy(dst_hbm.at[wid, 0], dsta_v, isema)
  pltpu.async_copy(dst_hbm.at[wid, 1], dstb_v, isemb)
  pltpu.async_copy(hp_sh.at[src_v.at[0]], rows0_v, gsem0)

  def body(i, _):
    j0 = 2 * i
    j1 = j0 + 1
    j2 = jnp.minimum(j0 + 2, CH2 - 1)
    j3 = jnp.minimum(j0 + 3, CH2 - 1)
    pltpu.async_copy(hp_sh.at[src_v.at[j1]], rows1_v, gsem1)
    pltpu.make_async_copy(hp_sh.at[src_v.at[j0]], rows0_v, gsem0).wait()
    pltpu.make_async_copy(dst_hbm.at[wid, j0], dsta_v, isema).wait()
    pltpu.sync_copy(rows0_v, acc_sh.at[dsta_v], add=True)
    pltpu.async_copy(hp_sh.at[src_v.at[j2]], rows0_v, gsem0)
    pltpu.async_copy(dst_hbm.at[wid, j2], dsta_v, isema)
    pltpu.make_async_copy(hp_sh.at[src_v.at[j1]], rows1_v, gsem1).wait()
    pltpu.make_async_copy(dst_hbm.at[wid, j1], dstb_v, isemb).wait()
    pltpu.sync_copy(rows1_v, acc_sh.at[dstb_v], add=True)
    pltpu.async_copy(dst_hbm.at[wid, j3], dstb_v, isemb)
    return 0

  lax.fori_loop(0, CH2 // 2, body, 0)
  pltpu.make_async_copy(hp_sh.at[src_v.at[CH2 - 1]], rows0_v, gsem0).wait()
  pltpu.make_async_copy(dst_hbm.at[wid, CH2 - 1], dsta_v, isema).wait()
  pltpu.make_async_copy(dst_hbm.at[wid, CH2 - 1], dstb_v, isemb).wait()
  plsc.subcore_barrier()

  pltpu.sync_copy(acc_sh.at[sl], s_out.at[c, sl])


def _make_pass2():
  mesh = plsc.VectorSubcoreMesh(core_axis_name="c", subcore_axis_name="s")
  out_type = [jax.ShapeDtypeStruct((NC, NPAD, CP), jnp.float32)]
  scratch = [
      pltpu.VMEM((CH2, K), jnp.int32),
      pltpu.VMEM((K,), jnp.int32),
      pltpu.VMEM((K,), jnp.int32),
      pltpu.VMEM((K, CP), jnp.float32),
      pltpu.VMEM((K, CP), jnp.float32),
      pltpu.VMEM_SHARED((NPAD, CP), jnp.float32),
      pltpu.VMEM_SHARED((NPAD, CP), jnp.float32),
      pltpu.SemaphoreType.DMA,
      pltpu.SemaphoreType.DMA,
      pltpu.SemaphoreType.DMA,
      pltpu.SemaphoreType.DMA,
      pltpu.SemaphoreType.DMA,
  ]
  return pl.kernel(
      _pass2_body, out_type=out_type, mesh=mesh, scratch_types=scratch,
      compiler_params=pltpu.CompilerParams(use_tc_tiling_on_sc=False))


def _tc1_body(s1a, s1b, dega, degb, x, w1l, w1r, b1, w2lp, h_out, hp_out):
  d = jnp.maximum(dega[0] + degb[0], 1.0)
  agg = jnp.concatenate([s1a[0], s1b[0]], axis=1) / d
  h = agg @ w1l[...] + x[...] @ w1r[...] + b1[...]
  h = jnp.maximum(h, 0.0)
  h_out[...] = h
  hp_out[...] = h @ w2lp[...]


def _tc2_body(s2a, s2b, dega, degb, h, w2rp, b2p, out):
  d = jnp.maximum(dega[0] + degb[0], 1.0)
  logits = (s2a[0] + s2b[0]) / d + h[...] @ w2rp[...] + b2p[...]
  col = lax.broadcasted_iota(jnp.int32, logits.shape, 1)
  ml = jnp.where(col < C, logits, -1e30)
  m = jnp.max(ml, axis=-1, keepdims=True)
  lse = jnp.log(jnp.sum(jnp.exp(ml - m), axis=-1, keepdims=True)) + m
  out[...] = ml - lse


_BR = 1264


def _tc1(s1, deg, xpad, w1l, w1r, b1, w2lp):
  grid = (NPAD // _BR,)
  row = lambda i: (i, 0)
  rowa = lambda i: (0, i, 0)
  rowb = lambda i: (1, i, 0)
  full = lambda i: (0, 0)
  return pl.pallas_call(
      _tc1_body,
      grid=grid,
      in_specs=[
          pl.BlockSpec((1, _BR, DH), rowa), pl.BlockSpec((1, _BR, DH), rowb),
          pl.BlockSpec((1, _BR, 1), rowa), pl.BlockSpec((1, _BR, 1), rowb),
          pl.BlockSpec((_BR, D), row),
          pl.BlockSpec((D, H), full), pl.BlockSpec((D, H), full),
          pl.BlockSpec((1, H), full), pl.BlockSpec((H, CP), full),
      ],
      out_specs=[pl.BlockSpec((_BR, H), row), pl.BlockSpec((_BR, CP), row)],
      out_shape=[jax.ShapeDtypeStruct((NPAD, H), jnp.float32),
                 jax.ShapeDtypeStruct((NPAD, CP), jnp.float32)],
  )(s1, s1, deg, deg, xpad, w1l, w1r, b1, w2lp)


def _tc2(s2, deg, h, w2rp, b2p):
  grid = (NPAD // _BR,)
  row = lambda i: (i, 0)
  rowa = lambda i: (0, i, 0)
  rowb = lambda i: (1, i, 0)
  full = lambda i: (0, 0)
  return pl.pallas_call(
      _tc2_body,
      grid=grid,
      in_specs=[
          pl.BlockSpec((1, _BR, CP), rowa), pl.BlockSpec((1, _BR, CP), rowb),
          pl.BlockSpec((1, _BR, 1), rowa), pl.BlockSpec((1, _BR, 1), rowb),
          pl.BlockSpec((_BR, H), row),
          pl.BlockSpec((H, CP), full), pl.BlockSpec((1, CP), full),
      ],
      out_specs=pl.BlockSpec((_BR, CP), row),
      out_shape=jax.ShapeDtypeStruct((NPAD, CP), jnp.float32),
  )(s2, s2, deg, deg, h, w2rp, b2p)


def kernel(x, edge_index, W1l, W1r, b1, W2l, W2r, b2):
  src = edge_index[0].astype(jnp.int32)
  dst = edge_index[1].astype(jnp.int32)
  pad = EP - E
  srcp = jnp.concatenate([src, jnp.zeros((pad,), jnp.int32)])
  dstp = jnp.concatenate([dst, jnp.full((pad,), N, jnp.int32)])
  src1 = srcp.reshape(NS, CH1, K)
  dst1 = dstp.reshape(NS, CH1, K)
  src2 = srcp.reshape(NW, CH2, K)
  dst2 = dstp.reshape(NW, CH2, K)

  xpad = jnp.pad(x, ((0, NPAD - N), (0, 0)))

  zrows = jnp.zeros((RPT, DH), jnp.float32)
  zrows_c = jnp.zeros((RPT, CP), jnp.float32)
  zvec = jnp.zeros((RPT,), jnp.float32)
  ones = jnp.ones((K,), jnp.float32)

  s1, deg = _make_pass1()(src1, dst1, xpad, zrows, zvec, ones)
  deg = deg.reshape(NC, NPAD, 1)

  w2lp = jnp.pad(W2l, ((0, 0), (0, CP - C)))
  h, hp = _tc1(s1, deg, xpad, W1l, W1r, b1[None, :], w2lp)

  s2 = _make_pass2()(src2, dst2, hp, zrows_c)
  if isinstance(s2, (tuple, list)):
    s2 = s2[0]

  w2rp = jnp.pad(W2r, ((0, 0), (0, CP - C)))
  b2p = jnp.pad(b2, (0, CP - C))[None, :]
  out = _tc2(s2, deg, h, w2rp, b2p)
  return out[:N, :C]
